# baseline (speedup 1.0000x reference)
import functools

import jax
import jax.numpy as jnp
from jax import lax
from jax.experimental import pallas as pl
from jax.experimental.pallas import tpu as pltpu
from jax.experimental.pallas import tpu_sc as plsc

_N = 1048576
_OHEM = 3
_NC, _NS, _L = 2, 16, 16
_NW = _NC * _NS
_PER_W = _N // _NW
_ITERS = _PER_W // _L

_SP_C = (-1.7414117e-02, 8.2691424e-02, -1.9035463e-01, 3.1574753e-01,
         -4.9737328e-01, 9.9984771e-01, 1.4716139e-06)

@functools.cache
def _make_sc_reduce():
    mesh = plsc.VectorSubcoreMesh(core_axis_name="c", subcore_axis_name="s")
    return pl.kernel(
        _sc_reduce_body,
        out_type=(jax.ShapeDtypeStruct((_NW, _L), jnp.float32),
                  jax.ShapeDtypeStruct((_NW, _L), jnp.int32)),
        mesh=mesh,
        compiler_params=pltpu.CompilerParams(needs_layout_passes=False),
        scratch_types=[
            pltpu.VMEM((_PER_W,), jnp.float32),
            pltpu.VMEM((_PER_W,), jnp.int32),
            pltpu.VMEM((_L,), jnp.float32),
            pltpu.VMEM((_L,), jnp.int32),
        ],
    )


def _sc_reduce_body(d_hbm, label_hbm, out_nll, out_cnt, dbuf, lbuf, obuf_f, obuf_i):
    wid = lax.axis_index("s") * _NC + lax.axis_index("c")
    base = wid * _PER_W
    pltpu.sync_copy(d_hbm.at[pl.ds(base, _PER_W)], dbuf)
    pltpu.sync_copy(label_hbm.at[pl.ds(base, _PER_W)], lbuf)

    zf = jnp.zeros((_L,), jnp.float32)
    onei = jnp.full((_L,), 1, jnp.int32)
    zi = jnp.zeros((_L,), jnp.int32)

    def body(i, carry):
        accf, acci = carry
        d = dbuf[pl.ds(i * _L, _L)]
        lab = lbuf[pl.ds(i * _L, _L)]
        t = jnp.exp(-jnp.abs(d))
        sp = jnp.full((_L,), _SP_C[0], jnp.float32)
        for c in _SP_C[1:]:
            sp = sp * t + jnp.full((_L,), c, jnp.float32)
        isneg = lab == 0
        z = jnp.where(isneg, d, -d)
        nll = jnp.maximum(z, zf) + sp
        return accf + nll, acci + jnp.where(isneg, zi, onei)

    accf, acci = lax.fori_loop(
        0, _ITERS, body,
        (jnp.zeros((_L,), jnp.float32), jnp.zeros((_L,), jnp.int32)))
    obuf_f[...] = accf
    obuf_i[...] = acci
    pltpu.sync_copy(obuf_f, out_nll.at[wid])
    pltpu.sync_copy(obuf_i, out_cnt.at[wid])


_RB = _N // 128
_NBLK = 16
_RPB = _RB // _NBLK

_MININT = -2147483648
_MAXPOS = 2147483647


def _skey(score):
    b = lax.bitcast_convert_type(score, jnp.int32)
    return jnp.where(b >= 0, b, b ^ jnp.int32(_MAXPOS))


def _rare_body(p0_ref, p1_ref, lab_ref, out_ref, si, sf):
    p = pl.program_id(0)
    b = pl.program_id(1)
    lab = lab_ref[...]
    neg = lab == 0

    @pl.when((p == 0) & (b == 0))
    def _():
        si[0] = 0

    @pl.when(p == 0)
    def _():
        si[0] = si[0] + jnp.sum((lab != 0).astype(jnp.int32))

    @pl.when((p >= 1) & (p <= 33) & (b == 0))
    def _():
        k = si[0] * _OHEM

        @pl.when(p == 1)
        def _():
            si[2] = 0

        @pl.when(p >= 2)
        def _():
            prevbit = jnp.left_shift(jnp.int32(1), 33 - p)
            si[2] = jnp.where(si[1] >= k, si[2] | prevbit, si[2])
        si[1] = 0

    @pl.when((p >= 1) & (p <= 32))
    def _():
        bit = jnp.left_shift(jnp.int32(1), 32 - p)
        scand = (si[2] | bit) ^ jnp.int32(_MININT)
        skey = _skey(p1_ref[...])
        si[1] = si[1] + jnp.sum((neg & (skey >= scand)).astype(jnp.int32))

    @pl.when((p == 33) & (b == 0))
    def _():
        k = si[0] * _OHEM
        ts = si[2] ^ jnp.int32(_MININT)
        si[4] = jnp.where(k == 0, jnp.int32(_MININT), ts)
        si[3] = 0
        sf[0] = 0.0

    @pl.when(p == 33)
    def _():
        p0 = p0_ref[...]
        p1 = p1_ref[...]
        skey = _skey(p1)
        m = (skey >= si[4]) | (lab != 0)
        mx = jnp.maximum(p0, p1)
        lse = mx + jnp.log(jnp.exp(p0 - mx) + jnp.exp(p1 - mx))
        nll = lse - jnp.where(lab == 0, p0, p1)
        sf[0] = sf[0] + jnp.sum(jnp.where(m, nll, 0.0))
        si[3] = si[3] + jnp.sum(m.astype(jnp.int32))

        @pl.when(b == _NBLK - 1)
        def _():
            out_ref[0] = sf[0] / jnp.maximum(si[3], 1).astype(jnp.float32)


def _rare(pred, label):
    p0 = pred[:, 0].reshape(_RB, 128)
    p1 = pred[:, 1].reshape(_RB, 128)
    lab = label.reshape(_RB, 128)
    out = pl.pallas_call(
        _rare_body,
        grid=(34, _NBLK),
        in_specs=[pl.BlockSpec((_RPB, 128), lambda p, b: (b, 0))] * 3,
        out_specs=pl.BlockSpec(memory_space=pltpu.MemorySpace.SMEM),
        out_shape=jax.ShapeDtypeStruct((1,), jnp.float32),
        scratch_shapes=[pltpu.SMEM((8,), jnp.int32),
                        pltpu.SMEM((4,), jnp.float32)],
    )(p0, p1, lab)
    return out[0]


def kernel(pred, label):
    cols = jnp.broadcast_to(jnp.array([1, 0], jnp.int32), (_N, 2))
    g = jnp.take_along_axis(pred, cols, axis=1)
    d = (g[:, 0] - g[:, 1]).reshape(_N)
    nll_p, cnt_p = _make_sc_reduce()(d, label)
    sum_nll = jnp.sum(nll_p)
    pos_num = jnp.sum(cnt_p)
    n_neg = jnp.int32(_N) - pos_num
    return lax.cond(n_neg > pos_num * _OHEM,
                    lambda: _rare(pred, label),
                    lambda: sum_nll / jnp.float32(_N))


# speedup vs baseline: 7.1758x; 7.1758x over previous
import functools

import jax
import jax.numpy as jnp
from jax import lax
from jax.experimental import pallas as pl
from jax.experimental.pallas import tpu as pltpu
from jax.experimental.pallas import tpu_sc as plsc

_N = 1048576
_OHEM = 3
_NC, _NS, _L = 2, 16, 16
_NW = _NC * _NS
_PER_W = _N // _NW
_ITERS = _PER_W // _L

_SP_C = (-1.7414117e-02, 8.2691424e-02, -1.9035463e-01, 3.1574753e-01,
         -4.9737328e-01, 9.9984771e-01, 1.4716139e-06)

@functools.cache
def _make_sc_reduce():
    mesh = plsc.VectorSubcoreMesh(core_axis_name="c", subcore_axis_name="s")
    return pl.kernel(
        _sc_reduce_body,
        out_type=jax.ShapeDtypeStruct((_NW, 2 * _L), jnp.float32),
        mesh=mesh,
        compiler_params=pltpu.CompilerParams(needs_layout_passes=False),
        scratch_types=[
            pltpu.VMEM((_PER_W,), jnp.float32),
            pltpu.VMEM((_PER_W,), jnp.int32),
            pltpu.VMEM((2 * _L,), jnp.float32),
        ],
    )


def _sc_reduce_body(d_hbm, label_hbm, out_p, dbuf, lbuf, obuf):
    wid = lax.axis_index("s") * _NC + lax.axis_index("c")
    base = wid * _PER_W
    pltpu.sync_copy(d_hbm.at[pl.ds(base, _PER_W)], dbuf)
    pltpu.sync_copy(label_hbm.at[pl.ds(base, _PER_W)], lbuf)

    zf = jnp.zeros((_L,), jnp.float32)
    onef = jnp.full((_L,), 1.0, jnp.float32)

    def body(i, carry):
        accf, accc = carry
        d = dbuf[pl.ds(i * _L, _L)]
        lab = lbuf[pl.ds(i * _L, _L)]
        t = jnp.exp(-jnp.abs(d))
        sp = jnp.full((_L,), _SP_C[0], jnp.float32)
        for c in _SP_C[1:]:
            sp = sp * t + jnp.full((_L,), c, jnp.float32)
        isneg = lab == 0
        z = jnp.where(isneg, d, -d)
        nll = jnp.maximum(z, zf) + sp
        return accf + nll, accc + jnp.where(isneg, zf, onef)

    accf, accc = lax.fori_loop(
        0, _ITERS, body,
        (jnp.zeros((_L,), jnp.float32), jnp.zeros((_L,), jnp.float32)))
    obuf[pl.ds(0, _L)] = accf
    obuf[pl.ds(_L, _L)] = accc
    pltpu.sync_copy(obuf, out_p.at[wid])


_RB = _N // 128
_NBLK = 16
_RPB = _RB // _NBLK

_MININT = -2147483648
_MAXPOS = 2147483647


def _skey(score):
    b = lax.bitcast_convert_type(score, jnp.int32)
    return jnp.where(b >= 0, b, b ^ jnp.int32(_MAXPOS))


def _rare_body(p0_ref, p1_ref, lab_ref, out_ref, si, sf):
    p = pl.program_id(0)
    b = pl.program_id(1)
    lab = lab_ref[...]
    neg = lab == 0

    @pl.when((p == 0) & (b == 0))
    def _():
        si[0] = 0

    @pl.when(p == 0)
    def _():
        si[0] = si[0] + jnp.sum((lab != 0).astype(jnp.int32))

    @pl.when((p >= 1) & (p <= 33) & (b == 0))
    def _():
        k = si[0] * _OHEM

        @pl.when(p == 1)
        def _():
            si[2] = 0

        @pl.when(p >= 2)
        def _():
            prevbit = jnp.left_shift(jnp.int32(1), 33 - p)
            si[2] = jnp.where(si[1] >= k, si[2] | prevbit, si[2])
        si[1] = 0

    @pl.when((p >= 1) & (p <= 32))
    def _():
        bit = jnp.left_shift(jnp.int32(1), 32 - p)
        scand = (si[2] | bit) ^ jnp.int32(_MININT)
        skey = _skey(p1_ref[...])
        si[1] = si[1] + jnp.sum((neg & (skey >= scand)).astype(jnp.int32))

    @pl.when((p == 33) & (b == 0))
    def _():
        k = si[0] * _OHEM
        ts = si[2] ^ jnp.int32(_MININT)
        si[4] = jnp.where(k == 0, jnp.int32(_MININT), ts)
        si[3] = 0
        sf[0] = 0.0

    @pl.when(p == 33)
    def _():
        p0 = p0_ref[...]
        p1 = p1_ref[...]
        skey = _skey(p1)
        m = (skey >= si[4]) | (lab != 0)
        mx = jnp.maximum(p0, p1)
        lse = mx + jnp.log(jnp.exp(p0 - mx) + jnp.exp(p1 - mx))
        nll = lse - jnp.where(lab == 0, p0, p1)
        sf[0] = sf[0] + jnp.sum(jnp.where(m, nll, 0.0))
        si[3] = si[3] + jnp.sum(m.astype(jnp.int32))

        @pl.when(b == _NBLK - 1)
        def _():
            out_ref[0] = sf[0] / jnp.maximum(si[3], 1).astype(jnp.float32)


def _rare(pred, label):
    p0 = pred[:, 0].reshape(_RB, 128)
    p1 = pred[:, 1].reshape(_RB, 128)
    lab = label.reshape(_RB, 128)
    out = pl.pallas_call(
        _rare_body,
        grid=(34, _NBLK),
        in_specs=[pl.BlockSpec((_RPB, 128), lambda p, b: (b, 0))] * 3,
        out_specs=pl.BlockSpec(memory_space=pltpu.MemorySpace.SMEM),
        out_shape=jax.ShapeDtypeStruct((1,), jnp.float32),
        scratch_shapes=[pltpu.SMEM((8,), jnp.int32),
                        pltpu.SMEM((4,), jnp.float32)],
    )(p0, p1, lab)
    return out[0]


def kernel(pred, label):
    idx0 = jnp.zeros((_N, 1), jnp.int32)
    d = (jnp.take_along_axis(pred, idx0 + 1, axis=1)
         - jnp.take_along_axis(pred, idx0, axis=1)).reshape(_N)
    parts = _make_sc_reduce()(d, label)
    sums = jnp.sum(parts.reshape(_NW, 2, _L), axis=(0, 2))
    sum_nll = sums[0]
    pos_num = sums[1].astype(jnp.int32)
    n_neg = jnp.int32(_N) - pos_num
    return lax.cond(n_neg > pos_num * _OHEM,
                    lambda: _rare(pred, label),
                    lambda: sum_nll / jnp.float32(_N))


# speedup vs baseline: 7.2960x; 1.0167x over previous
import functools

import numpy as np

import jax
import jax.numpy as jnp
from jax import lax
from jax.experimental import pallas as pl
from jax.experimental.pallas import tpu as pltpu
from jax.experimental.pallas import tpu_sc as plsc

_N = 1048576
_OHEM = 3
_NC, _NS, _L = 2, 16, 16
_NW = _NC * _NS
_PER_W = _N // _NW
_ITERS = _PER_W // _L

_SP_C = (-1.7414117e-02, 8.2691424e-02, -1.9035463e-01, 3.1574753e-01,
         -4.9737328e-01, 9.9984771e-01, 1.4716139e-06)

@functools.cache
def _make_sc_reduce():
    mesh = plsc.VectorSubcoreMesh(core_axis_name="c", subcore_axis_name="s")
    return pl.kernel(
        _sc_reduce_body,
        out_type=jax.ShapeDtypeStruct((_NW, 2 * _L), jnp.float32),
        mesh=mesh,
        compiler_params=pltpu.CompilerParams(needs_layout_passes=False),
        scratch_types=[
            pltpu.VMEM((_PER_W,), jnp.float32),
            pltpu.VMEM((_PER_W,), jnp.float32),
            pltpu.VMEM((_PER_W,), jnp.int32),
            pltpu.VMEM((2 * _L,), jnp.float32),
        ],
    )


def _sc_reduce_body(p0_hbm, p1_hbm, label_hbm, out_p, p0buf, p1buf, lbuf, obuf):
    wid = lax.axis_index("s") * _NC + lax.axis_index("c")
    base = wid * _PER_W
    pltpu.sync_copy(p0_hbm.at[pl.ds(base, _PER_W)], p0buf)
    pltpu.sync_copy(p1_hbm.at[pl.ds(base, _PER_W)], p1buf)
    pltpu.sync_copy(label_hbm.at[pl.ds(base, _PER_W)], lbuf)

    zf = jnp.zeros((_L,), jnp.float32)
    onef = jnp.full((_L,), 1.0, jnp.float32)

    def body(i, carry):
        accf, accc = carry
        d = (p1buf[pl.ds(i * _L, _L)]
             - p0buf[pl.ds(i * _L, _L)])
        lab = lbuf[pl.ds(i * _L, _L)]
        t = jnp.exp(-jnp.abs(d))
        sp = jnp.full((_L,), _SP_C[0], jnp.float32)
        for c in _SP_C[1:]:
            sp = sp * t + jnp.full((_L,), c, jnp.float32)
        isneg = lab == 0
        z = jnp.where(isneg, d, -d)
        nll = jnp.maximum(z, zf) + sp
        return accf + nll, accc + jnp.where(isneg, zf, onef)

    accf, accc = lax.fori_loop(
        0, _ITERS, body,
        (jnp.zeros((_L,), jnp.float32), jnp.zeros((_L,), jnp.float32)))
    obuf[pl.ds(0, _L)] = accf
    obuf[pl.ds(_L, _L)] = accc
    pltpu.sync_copy(obuf, out_p.at[wid])


_RB = _N // 128
_NBLK = 16
_RPB = _RB // _NBLK

_MININT = -2147483648
_MAXPOS = 2147483647


def _skey(score):
    b = lax.bitcast_convert_type(score, jnp.int32)
    return jnp.where(b >= 0, b, b ^ jnp.int32(_MAXPOS))


def _rare_body(p0_ref, p1_ref, lab_ref, out_ref, si, sf):
    p = pl.program_id(0)
    b = pl.program_id(1)
    lab = lab_ref[...]
    neg = lab == 0

    @pl.when((p == 0) & (b == 0))
    def _():
        si[0] = 0

    @pl.when(p == 0)
    def _():
        si[0] = si[0] + jnp.sum((lab != 0).astype(jnp.int32))

    @pl.when((p >= 1) & (p <= 33) & (b == 0))
    def _():
        k = si[0] * _OHEM

        @pl.when(p == 1)
        def _():
            si[2] = 0

        @pl.when(p >= 2)
        def _():
            prevbit = jnp.left_shift(jnp.int32(1), 33 - p)
            si[2] = jnp.where(si[1] >= k, si[2] | prevbit, si[2])
        si[1] = 0

    @pl.when((p >= 1) & (p <= 32))
    def _():
        bit = jnp.left_shift(jnp.int32(1), 32 - p)
        scand = (si[2] | bit) ^ jnp.int32(_MININT)
        skey = _skey(p1_ref[...])
        si[1] = si[1] + jnp.sum((neg & (skey >= scand)).astype(jnp.int32))

    @pl.when((p == 33) & (b == 0))
    def _():
        k = si[0] * _OHEM
        ts = si[2] ^ jnp.int32(_MININT)
        si[4] = jnp.where(k == 0, jnp.int32(_MININT), ts)
        si[3] = 0
        sf[0] = 0.0

    @pl.when(p == 33)
    def _():
        p0 = p0_ref[...]
        p1 = p1_ref[...]
        skey = _skey(p1)
        m = (skey >= si[4]) | (lab != 0)
        mx = jnp.maximum(p0, p1)
        lse = mx + jnp.log(jnp.exp(p0 - mx) + jnp.exp(p1 - mx))
        nll = lse - jnp.where(lab == 0, p0, p1)
        sf[0] = sf[0] + jnp.sum(jnp.where(m, nll, 0.0))
        si[3] = si[3] + jnp.sum(m.astype(jnp.int32))

        @pl.when(b == _NBLK - 1)
        def _():
            out_ref[0] = sf[0] / jnp.maximum(si[3], 1).astype(jnp.float32)


def _rare(pred, label):
    p0 = pred[:, 0].reshape(_RB, 128)
    p1 = pred[:, 1].reshape(_RB, 128)
    lab = label.reshape(_RB, 128)
    out = pl.pallas_call(
        _rare_body,
        grid=(34, _NBLK),
        in_specs=[pl.BlockSpec((_RPB, 128), lambda p, b: (b, 0))] * 3,
        out_specs=pl.BlockSpec(memory_space=pltpu.MemorySpace.SMEM),
        out_shape=jax.ShapeDtypeStruct((1,), jnp.float32),
        scratch_shapes=[pltpu.SMEM((8,), jnp.int32),
                        pltpu.SMEM((4,), jnp.float32)],
    )(p0, p1, lab)
    return out[0]


def kernel(pred, label):
    idx0 = np.zeros((_N, 1), np.int32)
    idx1 = np.ones((_N, 1), np.int32)
    p0 = jnp.take_along_axis(pred, idx0, axis=1).reshape(_N)
    p1 = jnp.take_along_axis(pred, idx1, axis=1).reshape(_N)
    parts = _make_sc_reduce()(p0, p1, label)
    sums = jnp.sum(parts.reshape(_NW, 2, _L), axis=(0, 2))
    sum_nll = sums[0]
    pos_num = sums[1].astype(jnp.int32)
    n_neg = jnp.int32(_N) - pos_num
    return lax.cond(n_neg > pos_num * _OHEM,
                    lambda: _rare(pred, label),
                    lambda: sum_nll / jnp.float32(_N))
